# baseline (speedup 1.0000x reference)
import functools

import jax
import jax.numpy as jnp
from jax import lax
from jax.experimental import pallas as pl
from jax.experimental.pallas import tpu as pltpu
from jax.experimental.pallas import tpu_sc as plsc

NC, NS = 2, 16
NW = NC * NS
CH = 128


def _pad_rows(x, n):
    return jnp.pad(x, ((0, n - x.shape[0]),) + ((0, 0),) * (x.ndim - 1))


def _prep_kernel(wsp, wp1, wpt, wp2, wp3, wmv, wm1, wmt, wm2,
                 wit, wi1, wic, wi2, bip, wab, wa1, wef, wa2, bap,
                 psp, pt1, pt2, pmv, pmt, pit, pic, pab, pef):
    dot = functools.partial(jnp.dot, preferred_element_type=jnp.float32)
    psp[...] = dot(wsp[...], wp1[...])
    pt1[...] = dot(wpt[...], wp2[...])
    pt2[...] = dot(wpt[...], wp3[...])
    pmv[...] = dot(wmv[...], wm1[...])
    pmt[...] = dot(wmt[...], wm2[...])
    pit[...] = dot(wit[...], wi1[...])
    pic[...] = dot(wic[...], wi2[...]) + bip[...]
    pab[...] = dot(wab[...], wa1[...])
    pef[...] = dot(wef[...], wa2[...]) + bap[...]


def _dense_kernel(t1, t2, bs, mt, mp,
                  pt1, pt2, pmt,
                  ws1, bs1, ws2, bs2, wp4, bpp,
                  wq1, bq1, wq2, bq2, wm3, bmp,
                  pok_o, mov_o):
    f32 = jnp.float32
    dot = functools.partial(jnp.dot, preferred_element_type=f32)

    def onehot(ids, n):
        r = ids.shape[0]
        return (ids.reshape(r, 1) ==
                lax.broadcasted_iota(jnp.int32, (r, n), 1)).astype(f32)

    st = jnp.maximum(dot(bs[...], ws1[...]) + bs1[...], 0.0)
    st = dot(st, ws2[...]) + bs2[...]
    pok = dot(onehot(t1[0, 0], 32), pt1[...])
    pok += dot(onehot(t2[0, 0], 32), pt2[...])
    pok += dot(st, wp4[...]) + bpp[...]
    pok_o[...] = pok.reshape(pok_o.shape)

    pe = jnp.maximum(dot(mp[...], wq1[...]) + bq1[...], 0.0)
    pe = dot(pe, wq2[...]) + bq2[...]
    mov = dot(onehot(mt[0, 0], 32), pmt[...])
    mov += dot(pe, wm3[...]) + bmp[...]
    mov_o[...] = mov.reshape(mov_o.shape)


def _sc_add_store(g, t, out_hbm, base, rows, cols):
    nj = cols // 16

    def add_body(i, _):
        for j in range(nj):
            sl = pl.ds(j * 16, 16)
            g[i, sl] = g[i, sl] + t[i, sl]
        return 0

    lax.fori_loop(0, rows, add_body, 0)
    pltpu.sync_copy(g, out_hbm.at[pl.ds(base, rows)])


def _sc_main(sid_hbm, mv_hbm, iid_hbm, ic_hbm, aid_hbm, ef_hbm,
             psp_hbm, pmv_hbm, pit_hbm, pic_hbm, pab_hbm, pef_hbm,
             poktc_hbm, movtc_hbm,
             pok_out, mov_out, itm_out, abl_out,
             idx_v, idx2_v, g_p, t_p, g_m, t_m, g_a, t_a, sem, sem2):
    wid = lax.axis_index("s") * NC + lax.axis_index("c")
    n1 = sid_hbm.shape[0] // NW
    n2 = mv_hbm.shape[0] // NW

    def pok_body(k, _):
        base = wid * n1 + k * CH
        pltpu.sync_copy(sid_hbm.at[pl.ds(base, CH)], idx_v)
        cp = pltpu.async_copy(psp_hbm.at[idx_v], g_p, sem)
        pltpu.sync_copy(poktc_hbm.at[pl.ds(base, CH)], t_p)
        cp.wait()
        _sc_add_store(g_p, t_p, pok_out, base, CH, 128)
        return 0

    lax.fori_loop(0, n1 // CH, pok_body, 0)

    def mov_body(k, _):
        base = wid * n2 + k * CH
        pltpu.sync_copy(mv_hbm.at[pl.ds(base, CH)], idx_v)
        cp = pltpu.async_copy(pmv_hbm.at[idx_v], g_m, sem)
        pltpu.sync_copy(movtc_hbm.at[pl.ds(base, CH)], t_m)
        cp.wait()
        _sc_add_store(g_m, t_m, mov_out, base, CH, 64)
        return 0

    lax.fori_loop(0, n2 // CH, mov_body, 0)

    def itm_body(k, _):
        base = wid * n1 + k * CH
        pltpu.sync_copy(iid_hbm.at[pl.ds(base, CH)], idx_v)
        pltpu.sync_copy(ic_hbm.at[pl.ds(base, CH)], idx2_v)
        cp = pltpu.async_copy(pit_hbm.at[idx_v], g_a, sem)
        cp2 = pltpu.async_copy(pic_hbm.at[idx2_v], t_a, sem2)
        cp.wait()
        cp2.wait()
        _sc_add_store(g_a, t_a, itm_out, base, CH, 32)
        return 0

    lax.fori_loop(0, n1 // CH, itm_body, 0)

    def abl_body(k, _):
        base = wid * n1 + k * CH
        pltpu.sync_copy(aid_hbm.at[pl.ds(base, CH)], idx_v)
        pltpu.sync_copy(ef_hbm.at[pl.ds(base, CH)], idx2_v)
        cp = pltpu.async_copy(pab_hbm.at[idx_v], g_a, sem)
        cp2 = pltpu.async_copy(pef_hbm.at[idx2_v], t_a, sem2)
        cp.wait()
        cp2.wait()
        _sc_add_store(g_a, t_a, abl_out, base, CH, 32)
        return 0

    lax.fori_loop(0, n1 // CH, abl_body, 0)


def kernel(species_ids, type1_ids, type2_ids, base_stats, move_ids,
           move_type_ids, move_properties, item_ids, item_category_ids,
           ability_ids, effect_ids, W_species, W_ptype, W_stat1, b_stat1,
           W_stat2, b_stat2, W_pproj, b_pproj, W_move, W_mtype, W_prop1,
           b_prop1, W_prop2, b_prop2, W_mproj, b_mproj, W_item, W_icat,
           W_iproj, b_iproj, W_ability, W_effect, W_aproj, b_aproj):
    B, T = species_ids.shape
    M = move_ids.shape[2]
    N = B * T
    NM2 = N * M
    R = 512
    G = N // R
    RM = R * M
    pd, md, idm, ad = 128, 64, 32, 32
    f32 = jnp.float32
    row = lambda v: v.reshape(1, -1)

    prep_in = [
        _pad_rows(W_species, 2048), W_pproj[0:128],
        _pad_rows(W_ptype, 32), W_pproj[128:144], W_pproj[144:160],
        _pad_rows(W_move, 1024), W_mproj[0:64],
        _pad_rows(W_mtype, 32), W_mproj[64:80],
        _pad_rows(W_item, 512), W_iproj[0:32],
        _pad_rows(W_icat, 32), W_iproj[32:40], row(b_iproj),
        _pad_rows(W_ability, 512), W_aproj[0:32],
        _pad_rows(W_effect, 32), W_aproj[32:40], row(b_aproj),
    ]
    prep_out = [
        jax.ShapeDtypeStruct((2048, pd), f32),
        jax.ShapeDtypeStruct((32, pd), f32),
        jax.ShapeDtypeStruct((32, pd), f32),
        jax.ShapeDtypeStruct((1024, md), f32),
        jax.ShapeDtypeStruct((32, md), f32),
        jax.ShapeDtypeStruct((512, idm), f32),
        jax.ShapeDtypeStruct((32, idm), f32),
        jax.ShapeDtypeStruct((512, ad), f32),
        jax.ShapeDtypeStruct((32, ad), f32),
    ]
    (psp, pt1, pt2, pmv, pmt, pit, pic, pab, pef) = pl.pallas_call(
        _prep_kernel, out_shape=prep_out)(*prep_in)

    t1 = type1_ids.reshape(G, 1, R).astype(jnp.int32)
    t2 = type2_ids.reshape(G, 1, R).astype(jnp.int32)
    mt = move_type_ids.reshape(G, 1, RM).astype(jnp.int32)
    bs = jnp.pad(base_stats.reshape(N, 6), ((0, 0), (0, 2))).reshape(G, R, 8)
    mp = jnp.pad(move_properties.reshape(NM2, 20),
                 ((0, 0), (0, 12))).reshape(G, RM, 32)
    ws1 = _pad_rows(W_stat1, 8)
    wq1 = _pad_rows(W_prop1, 32)

    idx_spec = lambda r: pl.BlockSpec((1, 1, r), lambda i: (i, 0, 0))
    dense_spec = lambda r, c: pl.BlockSpec((1, r, c), lambda i: (i, 0, 0))
    full = lambda *s: pl.BlockSpec(s, lambda i: (0,) * len(s))

    in_specs = (
        [idx_spec(R), idx_spec(R), pl.BlockSpec((1, R, 8), lambda i: (i, 0, 0)),
         idx_spec(RM), pl.BlockSpec((1, RM, 32), lambda i: (i, 0, 0))]
        + [full(32, pd), full(32, pd), full(32, md)]
        + [full(8, 32), full(1, 32), full(32, 32), full(1, 32),
           full(32, pd), full(1, pd),
           full(32, 32), full(1, 32), full(32, 32), full(1, 32),
           full(32, md), full(1, md)]
    )
    pok_tc, mov_tc = pl.pallas_call(
        _dense_kernel,
        grid=(G,),
        in_specs=in_specs,
        out_specs=[dense_spec(R, pd), dense_spec(RM, md)],
        out_shape=[jax.ShapeDtypeStruct((G, R, pd), f32),
                   jax.ShapeDtypeStruct((G, RM, md), f32)],
    )(t1, t2, bs, mt, mp, pt1, pt2, pmt,
      ws1, row(b_stat1), W_stat2, row(b_stat2), W_pproj[160:192],
      row(b_pproj), wq1, row(b_prop1), W_prop2, row(b_prop2),
      W_mproj[80:112], row(b_mproj))

    mesh = plsc.VectorSubcoreMesh(core_axis_name="c", subcore_axis_name="s")
    sc = pl.kernel(
        _sc_main,
        out_type=[jax.ShapeDtypeStruct((N, pd), f32),
                  jax.ShapeDtypeStruct((NM2, md), f32),
                  jax.ShapeDtypeStruct((N, idm), f32),
                  jax.ShapeDtypeStruct((N, ad), f32)],
        mesh=mesh,
        compiler_params=pltpu.CompilerParams(use_tc_tiling_on_sc=False),
        scratch_types=[
            pltpu.VMEM((CH,), jnp.int32),
            pltpu.VMEM((CH,), jnp.int32),
            pltpu.VMEM((CH, pd), f32),
            pltpu.VMEM((CH, pd), f32),
            pltpu.VMEM((CH, md), f32),
            pltpu.VMEM((CH, md), f32),
            pltpu.VMEM((CH, idm), f32),
            pltpu.VMEM((CH, idm), f32),
            pltpu.SemaphoreType.DMA,
            pltpu.SemaphoreType.DMA,
        ],
    )
    pok, mov, itm, abl = sc(
        species_ids.reshape(N).astype(jnp.int32),
        move_ids.reshape(NM2).astype(jnp.int32),
        item_ids.reshape(N).astype(jnp.int32),
        item_category_ids.reshape(N).astype(jnp.int32),
        ability_ids.reshape(N).astype(jnp.int32),
        effect_ids.reshape(N).astype(jnp.int32),
        psp, pmv, pit, pic, pab, pef,
        pok_tc.reshape(N, pd), mov_tc.reshape(NM2, md))

    return (pok.reshape(B, T, pd), mov.reshape(B, T, M, md),
            itm.reshape(B, T, idm), abl.reshape(B, T, ad))


# speedup vs baseline: 1.0503x; 1.0503x over previous
import functools

import jax
import jax.numpy as jnp
from jax import lax
from jax.experimental import pallas as pl
from jax.experimental.pallas import tpu as pltpu
from jax.experimental.pallas import tpu_sc as plsc

NC, NS = 2, 16
NW = NC * NS
CH = 128


def _pad_rows(x, n):
    return jnp.pad(x, ((0, n - x.shape[0]),) + ((0, 0),) * (x.ndim - 1))


def _prep_kernel(wsp, wp1, wpt, wp2, wp3, wmv, wm1, wmt, wm2,
                 wit, wi1, wic, wi2, bip, wab, wa1, wef, wa2, bap,
                 psp, pt1, pt2, pmv, pmt, pit, pic, pab, pef):
    dot = functools.partial(jnp.dot, preferred_element_type=jnp.float32)
    psp[...] = dot(wsp[...], wp1[...])
    pt1[...] = dot(wpt[...], wp2[...])
    pt2[...] = dot(wpt[...], wp3[...])
    pmv[...] = dot(wmv[...], wm1[...])
    pmt[...] = dot(wmt[...], wm2[...])
    pit[...] = dot(wit[...], wi1[...])
    pic[...] = dot(wic[...], wi2[...]) + bip[...]
    pab[...] = dot(wab[...], wa1[...])
    pef[...] = dot(wef[...], wa2[...]) + bap[...]


def _dense_kernel(t1, t2, bs, mt, mp,
                  pt1, pt2, pmt,
                  ws1, bs1, ws2, bs2, wp4, bpp,
                  wq1, bq1, wq2, bq2, wm3, bmp,
                  pok_o, mov_o):
    f32 = jnp.float32
    dot = functools.partial(jnp.dot, preferred_element_type=f32)

    def onehot(ids, n):
        r = ids.shape[0]
        return (ids.reshape(r, 1) ==
                lax.broadcasted_iota(jnp.int32, (r, n), 1)).astype(f32)

    st = jnp.maximum(dot(bs[...], ws1[...]) + bs1[...], 0.0)
    st = dot(st, ws2[...]) + bs2[...]
    pok = dot(onehot(t1[0, 0], 32), pt1[...])
    pok += dot(onehot(t2[0, 0], 32), pt2[...])
    pok += dot(st, wp4[...]) + bpp[...]
    pok_o[...] = pok.reshape(pok_o.shape)

    pe = jnp.maximum(dot(mp[...], wq1[...]) + bq1[...], 0.0)
    pe = dot(pe, wq2[...]) + bq2[...]
    mov = dot(onehot(mt[0, 0], 32), pmt[...])
    mov += dot(pe, wm3[...]) + bmp[...]
    mov_o[...] = mov.reshape(mov_o.shape)


CHP = 96
CHM = 128
CHA = 128


def _sc_main(sid_hbm, mv_hbm, iid_hbm, ic_hbm, aid_hbm, ef_hbm,
             psp_hbm, pmv_hbm, pit_hbm, pic_hbm, pab_hbm, pef_hbm,
             poktc_hbm, movtc_hbm,
             pok_out, mov_out, itm_out, abl_out,
             idxp, idxm, idxa, idxb,
             g0, g1, t0, t1, mg0, mg1, mt0, mt1,
             ag0, ag1, ah0, ah1,
             sg0, sg1, st0, st1, so0, so1, sh0, sh1):
    wid = lax.axis_index("s") * NC + lax.axis_index("c")
    n1 = sid_hbm.shape[0] // NW
    n2 = mv_hbm.shape[0] // NW

    pltpu.sync_copy(sid_hbm.at[pl.ds(wid * n1, n1)], idxp)
    bufs_p = ((g0, t0, sg0, st0, so0), (g1, t1, sg1, st1, so1))

    def pok_pair(kk, _):
        cps = []
        for b, (g, t, sg, st, so) in enumerate(bufs_p):
            k = kk * 2 + b
            base = wid * n1 + k * CHP
            cg = pltpu.async_copy(psp_hbm.at[idxp.at[pl.ds(k * CHP, CHP)]],
                                  g, sg)
            ct = pltpu.async_copy(poktc_hbm.at[pl.ds(base * 128, CHP * 128)],
                                  t, st)
            cps.append((cg, ct))
        outs = []
        for b, (g, t, sg, st, so) in enumerate(bufs_p):
            k = kk * 2 + b
            base = wid * n1 + k * CHP
            cps[b][0].wait()
            cps[b][1].wait()

            def add_body(i, _):
                for j in range(8):
                    sl = pl.ds(i * 128 + j * 16, 16)
                    t[sl] = t[sl] + g[i, pl.ds(j * 16, 16)]
                return 0

            lax.fori_loop(0, CHP, add_body, 0)
            outs.append(pltpu.async_copy(
                t, pok_out.at[pl.ds(base * 128, CHP * 128)], so))
        for co in outs:
            co.wait()
        return 0

    lax.fori_loop(0, n1 // CHP // 2, pok_pair, 0)

    pltpu.sync_copy(mv_hbm.at[pl.ds(wid * n2, n2)], idxm)
    bufs_m = ((mg0, mt0, sg0, st0, so0), (mg1, mt1, sg1, st1, so1))

    def mov_pair(kk, _):
        cps = []
        for b, (g, t, sg, st, so) in enumerate(bufs_m):
            k = kk * 2 + b
            base = wid * n2 + k * CHM
            cg = pltpu.async_copy(pmv_hbm.at[idxm.at[pl.ds(k * CHM, CHM)]],
                                  g, sg)
            ct = pltpu.async_copy(movtc_hbm.at[pl.ds(base * 64, CHM * 64)],
                                  t, st)
            cps.append((cg, ct))
        outs = []
        for b, (g, t, sg, st, so) in enumerate(bufs_m):
            k = kk * 2 + b
            base = wid * n2 + k * CHM
            cps[b][0].wait()
            cps[b][1].wait()

            def add_body(i, _):
                for j in range(4):
                    sl = pl.ds(i * 64 + j * 16, 16)
                    t[sl] = t[sl] + g[i, pl.ds(j * 16, 16)]
                return 0

            lax.fori_loop(0, CHM, add_body, 0)
            outs.append(pltpu.async_copy(
                t, mov_out.at[pl.ds(base * 64, CHM * 64)], so))
        for co in outs:
            co.wait()
        return 0

    lax.fori_loop(0, n2 // CHM // 2, mov_pair, 0)

    def pair_family(idA_hbm, idB_hbm, tabA, tabB, out_hbm):
        pltpu.sync_copy(idA_hbm.at[pl.ds(wid * n1, n1)], idxa)
        pltpu.sync_copy(idB_hbm.at[pl.ds(wid * n1, n1)], idxb)
        bufs = ((ag0, ah0, sg0, sh0, so0), (ag1, ah1, sg1, sh1, so1))

        def body(kk, _):
            cps = []
            for b, (ga, ha, sg, sh, so) in enumerate(bufs):
                k = kk * 2 + b
                ca = pltpu.async_copy(
                    tabA.at[idxa.at[pl.ds(k * CHA, CHA)]], ga, sg)
                cb = pltpu.async_copy(
                    tabB.at[idxb.at[pl.ds(k * CHA, CHA)]], ha, sh)
                cps.append((ca, cb))
            outs = []
            for b, (ga, ha, sg, sh, so) in enumerate(bufs):
                k = kk * 2 + b
                base = wid * n1 + k * CHA
                cps[b][0].wait()
                cps[b][1].wait()

                def add_body(i, _):
                    for j in range(2):
                        sl = pl.ds(j * 16, 16)
                        ga[i, sl] = ga[i, sl] + ha[i, sl]
                    return 0

                lax.fori_loop(0, CHA, add_body, 0)
                outs.append(pltpu.async_copy(
                    ga, out_hbm.at[pl.ds(base, CHA)], so))
            for co in outs:
                co.wait()
            return 0

        lax.fori_loop(0, n1 // CHA // 2, body, 0)

    pair_family(iid_hbm, ic_hbm, pit_hbm, pic_hbm, itm_out)
    pair_family(aid_hbm, ef_hbm, pab_hbm, pef_hbm, abl_out)


def kernel(species_ids, type1_ids, type2_ids, base_stats, move_ids,
           move_type_ids, move_properties, item_ids, item_category_ids,
           ability_ids, effect_ids, W_species, W_ptype, W_stat1, b_stat1,
           W_stat2, b_stat2, W_pproj, b_pproj, W_move, W_mtype, W_prop1,
           b_prop1, W_prop2, b_prop2, W_mproj, b_mproj, W_item, W_icat,
           W_iproj, b_iproj, W_ability, W_effect, W_aproj, b_aproj):
    B, T = species_ids.shape
    M = move_ids.shape[2]
    N = B * T
    NM2 = N * M
    R = 512
    G = N // R
    RM = R * M
    pd, md, idm, ad = 128, 64, 32, 32
    f32 = jnp.float32
    row = lambda v: v.reshape(1, -1)

    prep_in = [
        _pad_rows(W_species, 2048), W_pproj[0:128],
        _pad_rows(W_ptype, 32), W_pproj[128:144], W_pproj[144:160],
        _pad_rows(W_move, 1024), W_mproj[0:64],
        _pad_rows(W_mtype, 32), W_mproj[64:80],
        _pad_rows(W_item, 512), W_iproj[0:32],
        _pad_rows(W_icat, 32), W_iproj[32:40], row(b_iproj),
        _pad_rows(W_ability, 512), W_aproj[0:32],
        _pad_rows(W_effect, 32), W_aproj[32:40], row(b_aproj),
    ]
    prep_out = [
        jax.ShapeDtypeStruct((2048, pd), f32),
        jax.ShapeDtypeStruct((32, pd), f32),
        jax.ShapeDtypeStruct((32, pd), f32),
        jax.ShapeDtypeStruct((1024, md), f32),
        jax.ShapeDtypeStruct((32, md), f32),
        jax.ShapeDtypeStruct((512, idm), f32),
        jax.ShapeDtypeStruct((32, idm), f32),
        jax.ShapeDtypeStruct((512, ad), f32),
        jax.ShapeDtypeStruct((32, ad), f32),
    ]
    (psp, pt1, pt2, pmv, pmt, pit, pic, pab, pef) = pl.pallas_call(
        _prep_kernel, out_shape=prep_out)(*prep_in)

    t1 = type1_ids.reshape(G, 1, R).astype(jnp.int32)
    t2 = type2_ids.reshape(G, 1, R).astype(jnp.int32)
    mt = move_type_ids.reshape(G, 1, RM).astype(jnp.int32)
    bs = jnp.pad(base_stats.reshape(N, 6), ((0, 0), (0, 2))).reshape(G, R, 8)
    mp = jnp.pad(move_properties.reshape(NM2, 20),
                 ((0, 0), (0, 12))).reshape(G, RM, 32)
    ws1 = _pad_rows(W_stat1, 8)
    wq1 = _pad_rows(W_prop1, 32)

    idx_spec = lambda r: pl.BlockSpec((1, 1, r), lambda i: (i, 0, 0))
    dense_spec = lambda r, c: pl.BlockSpec((1, r, c), lambda i: (i, 0, 0))
    full = lambda *s: pl.BlockSpec(s, lambda i: (0,) * len(s))

    in_specs = (
        [idx_spec(R), idx_spec(R), pl.BlockSpec((1, R, 8), lambda i: (i, 0, 0)),
         idx_spec(RM), pl.BlockSpec((1, RM, 32), lambda i: (i, 0, 0))]
        + [full(32, pd), full(32, pd), full(32, md)]
        + [full(8, 32), full(1, 32), full(32, 32), full(1, 32),
           full(32, pd), full(1, pd),
           full(32, 32), full(1, 32), full(32, 32), full(1, 32),
           full(32, md), full(1, md)]
    )
    pok_tc, mov_tc = pl.pallas_call(
        _dense_kernel,
        grid=(G,),
        in_specs=in_specs,
        out_specs=[dense_spec(R, pd), dense_spec(RM, md)],
        out_shape=[jax.ShapeDtypeStruct((G, R, pd), f32),
                   jax.ShapeDtypeStruct((G, RM, md), f32)],
    )(t1, t2, bs, mt, mp, pt1, pt2, pmt,
      ws1, row(b_stat1), W_stat2, row(b_stat2), W_pproj[160:192],
      row(b_pproj), wq1, row(b_prop1), W_prop2, row(b_prop2),
      W_mproj[80:112], row(b_mproj))

    n1t = N // NW
    n2t = NM2 // NW
    mesh = plsc.VectorSubcoreMesh(core_axis_name="c", subcore_axis_name="s")
    sc = pl.kernel(
        _sc_main,
        out_type=[jax.ShapeDtypeStruct((N * pd,), f32),
                  jax.ShapeDtypeStruct((NM2 * md,), f32),
                  jax.ShapeDtypeStruct((N, idm), f32),
                  jax.ShapeDtypeStruct((N, ad), f32)],
        mesh=mesh,
        compiler_params=pltpu.CompilerParams(use_tc_tiling_on_sc=False),
        scratch_types=[
            pltpu.VMEM((n1t,), jnp.int32),
            pltpu.VMEM((n2t,), jnp.int32),
            pltpu.VMEM((n1t,), jnp.int32),
            pltpu.VMEM((n1t,), jnp.int32),
            pltpu.VMEM((CHP, pd), f32),
            pltpu.VMEM((CHP, pd), f32),
            pltpu.VMEM((CHP * pd,), f32),
            pltpu.VMEM((CHP * pd,), f32),
            pltpu.VMEM((CHM, md), f32),
            pltpu.VMEM((CHM, md), f32),
            pltpu.VMEM((CHM * md,), f32),
            pltpu.VMEM((CHM * md,), f32),
            pltpu.VMEM((CHA, idm), f32),
            pltpu.VMEM((CHA, idm), f32),
            pltpu.VMEM((CHA, idm), f32),
            pltpu.VMEM((CHA, idm), f32),
        ] + [pltpu.SemaphoreType.DMA] * 8,
    )
    pok, mov, itm, abl = sc(
        species_ids.reshape(N).astype(jnp.int32),
        move_ids.reshape(NM2).astype(jnp.int32),
        item_ids.reshape(N).astype(jnp.int32),
        item_category_ids.reshape(N).astype(jnp.int32),
        ability_ids.reshape(N).astype(jnp.int32),
        effect_ids.reshape(N).astype(jnp.int32),
        psp, pmv, pit, pic, pab, pef,
        pok_tc.reshape(N * pd), mov_tc.reshape(NM2 * md))

    return (pok.reshape(B, T, pd), mov.reshape(B, T, M, md),
            itm.reshape(B, T, idm), abl.reshape(B, T, ad))
